# Initial kernel scaffold; baseline (speedup 1.0000x reference)
import jax
import jax.numpy as jnp
from jax.experimental import pallas as pl


def kernel(x, edge_index, W1, a_s1, a_d1, b1, W2, a_s2, a_d2, b2):
    raise NotImplementedError("write your pallas kernel here")


# speedup vs baseline: 43.0585x; 43.0585x over previous
import functools

import jax
import jax.numpy as jnp
from jax import lax
from jax.experimental import pallas as pl
from jax.experimental.pallas import tpu as pltpu
from jax.experimental.pallas import tpu_sc as plsc

N = 10000
NP = 10240
E = 320000
EPP = 331776
EPT = EPP // 32
BLK = 128
F_IN = 128
H1 = 8
C1 = 8
F1 = H1 * C1
C2 = 16
AW1 = 80
AW2 = 32
ROWS_PT = NP // 16


def _mm1_body(x_ref, w_ref, as_ref, ad_ref, h_ref, asrc_ref, adst_ref):
    h = jnp.dot(x_ref[...], w_ref[...], preferred_element_type=jnp.float32)
    h_ref[...] = h
    asrc_ref[...] = jnp.dot(h, as_ref[...], preferred_element_type=jnp.float32)
    adst_ref[...] = jnp.dot(h, ad_ref[...], preferred_element_type=jnp.float32)


def _stage1(x_pad, W1, AS1, AD1):
    blk = NP // 8
    return pl.pallas_call(
        _mm1_body,
        grid=(8,),
        in_specs=[
            pl.BlockSpec((blk, F_IN), lambda i: (i, 0)),
            pl.BlockSpec((F_IN, F1), lambda i: (0, 0)),
            pl.BlockSpec((F1, 16), lambda i: (0, 0)),
            pl.BlockSpec((F1, 16), lambda i: (0, 0)),
        ],
        out_specs=[
            pl.BlockSpec((blk, F1), lambda i: (i, 0)),
            pl.BlockSpec((blk, 16), lambda i: (i, 0)),
            pl.BlockSpec((blk, 16), lambda i: (i, 0)),
        ],
        out_shape=[
            jax.ShapeDtypeStruct((NP, F1), jnp.float32),
            jax.ShapeDtypeStruct((NP, 16), jnp.float32),
            jax.ShapeDtypeStruct((NP, 16), jnp.float32),
        ],
    )(x_pad, W1, AS1, AD1)


def _dgather(v, idx):
    return lax.gather(
        v, idx[:, None],
        lax.GatherDimensionNumbers(
            offset_dims=(), collapsed_slice_dims=(0,), start_index_map=(0,)),
        slice_sizes=(1,),
        mode=lax.GatherScatterMode.PROMISE_IN_BOUNDS)


def _make_edge_pass(fw, aw):
    nj = fw // 16
    mesh = plsc.VectorSubcoreMesh(core_axis_name="c", subcore_axis_name="s")

    @functools.partial(
        pl.kernel,
        mesh=mesh,
        compiler_params=pltpu.CompilerParams(use_tc_tiling_on_sc=False),
        out_type=jax.ShapeDtypeStruct((2, NP, aw), jnp.float32),
        scratch_types=[
            pltpu.VMEM((BLK,), jnp.int32),
            pltpu.VMEM((BLK,), jnp.int32),
            pltpu.VMEM((BLK, 16), jnp.float32),
            pltpu.VMEM((BLK, 16), jnp.float32),
            pltpu.VMEM((BLK, fw), jnp.float32),
            pltpu.VMEM((BLK, aw), jnp.float32),
            pltpu.VMEM_SHARED((NP, aw), jnp.float32),
            pltpu.SemaphoreType.DMA,
        ],
    )
    def edge_kernel(src_h, dst_h, as_h, ad_h, h_h, z_h, out_h,
                    sidx, didx, as_v, ad_v, h_v, msg_v, accum, sem):
        cid = lax.axis_index("c")
        sid = lax.axis_index("s")
        wid = cid * 16 + sid
        lo = sid * ROWS_PT

        pltpu.sync_copy(z_h.at[pl.ds(lo, ROWS_PT)], accum.at[pl.ds(lo, ROWS_PT)])
        plsc.subcore_barrier()

        def blk_body(b, carry):
            base = wid * EPT + b * BLK
            pltpu.sync_copy(src_h.at[pl.ds(base, BLK)], sidx)
            pltpu.sync_copy(dst_h.at[pl.ds(base, BLK)], didx)
            pltpu.async_copy(as_h.at[sidx], as_v, sem).wait()
            pltpu.async_copy(ad_h.at[didx], ad_v, sem).wait()
            pltpu.async_copy(h_h.at[sidx], h_v, sem).wait()

            lane = lax.iota(jnp.int32, 16)

            def edge_body(e, c2):
                ea = as_v[e] + ad_v[e]
                ea = jnp.where(ea >= 0, ea, 0.2 * ea)
                ex = jnp.exp(ea)
                for j in range(nj):
                    hvec = h_v[e, pl.ds(16 * j, 16)]
                    if nj > 1:
                        rep = _dgather(ex, jnp.where(lane < 8, 2 * j, 2 * j + 1))
                    else:
                        rep = ex
                    msg_v[e, pl.ds(16 * j, 16)] = hvec * rep
                msg_v[e, pl.ds(fw, 16)] = ex
                return c2

            lax.fori_loop(0, BLK, edge_body, 0)
            pltpu.sync_copy(msg_v, accum.at[didx], add=True)
            return carry

        lax.fori_loop(0, EPT // BLK, blk_body, 0)
        plsc.subcore_barrier()
        pltpu.sync_copy(accum.at[pl.ds(lo, ROWS_PT)],
                        out_h.at[cid, pl.ds(lo, ROWS_PT)])

    return edge_kernel


_edge_pass_l1 = _make_edge_pass(F1, AW1)
_edge_pass_l2 = _make_edge_pass(C2, AW2)


def _mid_body(p_ref, r_ref, b1_ref, w2_ref, as2_ref, ad2_ref,
              h2_ref, asr_ref, adr_ref):
    acc = p_ref[0] + p_ref[1]
    num = acc[:, :F1]
    den = acc[:, F1:F1 + H1]
    invr = jnp.dot(1.0 / (den + 1e-16), r_ref[...],
                   preferred_element_type=jnp.float32)
    hl = num * invr + b1_ref[...]
    hf = jnp.where(hl > 0, hl, jnp.exp(hl) - 1.0)
    h2 = jnp.dot(hf, w2_ref[...], preferred_element_type=jnp.float32)
    h2_ref[...] = h2
    asr_ref[...] = jnp.dot(h2, as2_ref[...], preferred_element_type=jnp.float32)
    adr_ref[...] = jnp.dot(h2, ad2_ref[...], preferred_element_type=jnp.float32)


def _stage3(part1, R8, b1_2d, W2, AS2, AD2):
    blk = NP // 8
    return pl.pallas_call(
        _mid_body,
        grid=(8,),
        in_specs=[
            pl.BlockSpec((2, blk, AW1), lambda i: (0, i, 0)),
            pl.BlockSpec((H1, F1), lambda i: (0, 0)),
            pl.BlockSpec((1, F1), lambda i: (0, 0)),
            pl.BlockSpec((F1, C2), lambda i: (0, 0)),
            pl.BlockSpec((C2, 16), lambda i: (0, 0)),
            pl.BlockSpec((C2, 16), lambda i: (0, 0)),
        ],
        out_specs=[
            pl.BlockSpec((blk, C2), lambda i: (i, 0)),
            pl.BlockSpec((blk, 16), lambda i: (i, 0)),
            pl.BlockSpec((blk, 16), lambda i: (i, 0)),
        ],
        out_shape=[
            jax.ShapeDtypeStruct((NP, C2), jnp.float32),
            jax.ShapeDtypeStruct((NP, 16), jnp.float32),
            jax.ShapeDtypeStruct((NP, 16), jnp.float32),
        ],
    )(part1, R8, b1_2d, W2, AS2, AD2)


def _fin_body(p_ref, b2_ref, out_ref):
    acc = p_ref[0] + p_ref[1]
    o = acc[:, :C2] / (acc[:, C2:2 * C2] + 1e-16) + b2_ref[...]
    m = jnp.max(o, axis=1, keepdims=True)
    out_ref[...] = o - m - jnp.log(jnp.sum(jnp.exp(o - m), axis=1,
                                           keepdims=True))


def _stage5(part2, b2_2d):
    blk = NP // 8
    return pl.pallas_call(
        _fin_body,
        grid=(8,),
        in_specs=[
            pl.BlockSpec((2, blk, AW2), lambda i: (0, i, 0)),
            pl.BlockSpec((1, C2), lambda i: (0, 0)),
        ],
        out_specs=pl.BlockSpec((blk, C2), lambda i: (i, 0)),
        out_shape=jax.ShapeDtypeStruct((NP, C2), jnp.float32),
    )(part2, b2_2d)


def kernel(x, edge_index, W1, a_s1, a_d1, b1, W2, a_s2, a_d2, b2):
    f32 = jnp.float32
    x_pad = jnp.pad(x, ((0, NP - N), (0, 0)))

    loops = jnp.arange(N, dtype=jnp.int32)
    pad = jnp.full((EPP - E - N,), N, dtype=jnp.int32)
    src = jnp.concatenate([edge_index[0].astype(jnp.int32), loops, pad])
    dst = jnp.concatenate([edge_index[1].astype(jnp.int32), loops, pad])

    rows = jnp.arange(F1)
    AS1 = jnp.zeros((F1, 16), f32).at[rows, rows // C1].set(a_s1.reshape(F1))
    AD1 = jnp.zeros((F1, 16), f32).at[rows, rows // C1].set(a_d1.reshape(F1))

    h1, asrc1, adst1 = _stage1(x_pad, W1, AS1, AD1)

    zeros1 = jnp.zeros((NP, AW1), f32)
    part1 = _edge_pass_l1(src, dst, asrc1, adst1, h1, zeros1)

    R8 = jnp.zeros((H1, F1), f32).at[jnp.arange(F1) // C1, jnp.arange(F1)].set(1.0)
    AS2 = jnp.broadcast_to(a_s2.reshape(C2, 1), (C2, 16)).astype(f32)
    AD2 = jnp.broadcast_to(a_d2.reshape(C2, 1), (C2, 16)).astype(f32)
    h2, asrc2, adst2 = _stage3(part1, R8, b1.reshape(1, F1), W2, AS2, AD2)

    zeros2 = jnp.zeros((NP, AW2), f32)
    part2 = _edge_pass_l2(src, dst, asrc2, adst2, h2, zeros2)

    out = _stage5(part2, b2.reshape(1, C2))
    return out[:N]


# speedup vs baseline: 63.7084x; 1.4796x over previous
import functools

import jax
import jax.numpy as jnp
from jax import lax
from jax.experimental import pallas as pl
from jax.experimental.pallas import tpu as pltpu
from jax.experimental.pallas import tpu_sc as plsc

N = 10000
NP = 10240
E = 320000
EPP = 335872
EPT = EPP // 32
BLK = 128
EALLOC = EPP + BLK
F_IN = 128
H1 = 8
C1 = 8
F1 = H1 * C1
C2 = 16
AW1 = 80
AW2 = 32
ROWS_PT = NP // 16


def _mm1_body(x_ref, w_ref, as_ref, ad_ref, h_ref, asrc_ref, adst_ref):
    h = jnp.dot(x_ref[...], w_ref[...], preferred_element_type=jnp.float32)
    h_ref[...] = h
    asrc_ref[...] = jnp.dot(h, as_ref[...], preferred_element_type=jnp.float32)
    adst_ref[...] = jnp.dot(h, ad_ref[...], preferred_element_type=jnp.float32)


def _stage1(x_pad, W1, AS1, AD1):
    blk = NP // 8
    return pl.pallas_call(
        _mm1_body,
        grid=(8,),
        in_specs=[
            pl.BlockSpec((blk, F_IN), lambda i: (i, 0)),
            pl.BlockSpec((F_IN, F1), lambda i: (0, 0)),
            pl.BlockSpec((F1, 16), lambda i: (0, 0)),
            pl.BlockSpec((F1, 16), lambda i: (0, 0)),
        ],
        out_specs=[
            pl.BlockSpec((blk, F1), lambda i: (i, 0)),
            pl.BlockSpec((blk, 16), lambda i: (i, 0)),
            pl.BlockSpec((blk, 16), lambda i: (i, 0)),
        ],
        out_shape=[
            jax.ShapeDtypeStruct((NP, F1), jnp.float32),
            jax.ShapeDtypeStruct((NP, 16), jnp.float32),
            jax.ShapeDtypeStruct((NP, 16), jnp.float32),
        ],
    )(x_pad, W1, AS1, AD1)


def _dgather(v, idx):
    return lax.gather(
        v, idx[:, None],
        lax.GatherDimensionNumbers(
            offset_dims=(), collapsed_slice_dims=(0,), start_index_map=(0,)),
        slice_sizes=(1,),
        mode=lax.GatherScatterMode.PROMISE_IN_BOUNDS)


def _make_edge_pass(fw, aw):
    nj = fw // 16
    mesh = plsc.VectorSubcoreMesh(core_axis_name="c", subcore_axis_name="s")

    buf_types = []
    for _ in range(2):
        buf_types += [
            pltpu.VMEM((BLK,), jnp.int32),
            pltpu.VMEM((BLK,), jnp.int32),
            pltpu.VMEM((BLK, 16), jnp.float32),
            pltpu.VMEM((BLK, 16), jnp.float32),
            pltpu.VMEM((BLK, fw), jnp.float32),
            pltpu.VMEM((BLK, aw), jnp.float32),
            pltpu.SemaphoreType.DMA,
        ]

    @functools.partial(
        pl.kernel,
        mesh=mesh,
        compiler_params=pltpu.CompilerParams(use_tc_tiling_on_sc=False),
        out_type=jax.ShapeDtypeStruct((2, NP, aw), jnp.float32),
        scratch_types=buf_types + [pltpu.VMEM_SHARED((NP, aw), jnp.float32)],
    )
    def edge_kernel(src_h, dst_h, as_h, ad_h, h_h, z_h, out_h, *scr):
        bufs = (scr[0:7], scr[7:14])
        accum = scr[14]
        cid = lax.axis_index("c")
        sid = lax.axis_index("s")
        wid = cid * 16 + sid
        lo = sid * ROWS_PT

        def fire(buf, g):
            sidx, didx, as_v, ad_v, h_v, _, sem = buf
            base = wid * EPT + g * BLK
            pltpu.sync_copy(src_h.at[pl.ds(base, BLK)], sidx)
            pltpu.sync_copy(dst_h.at[pl.ds(base, BLK)], didx)
            pltpu.async_copy(as_h.at[sidx], as_v, sem)
            pltpu.async_copy(ad_h.at[didx], ad_v, sem)
            pltpu.async_copy(h_h.at[sidx], h_v, sem)

        def drain(buf):
            sidx, didx, as_v, ad_v, h_v, _, sem = buf
            pltpu.make_async_copy(as_h.at[sidx], as_v, sem).wait()
            pltpu.make_async_copy(ad_h.at[didx], ad_v, sem).wait()
            pltpu.make_async_copy(h_h.at[sidx], h_v, sem).wait()

        def compute_scatter(buf):
            _, didx, as_v, ad_v, h_v, msg_v, _ = buf
            lane = lax.iota(jnp.int32, 16)

            def edge_body(e, c2):
                ea = as_v[e] + ad_v[e]
                ea = jnp.where(ea >= 0, ea, 0.2 * ea)
                ex = jnp.exp(ea)
                for j in range(nj):
                    hvec = h_v[e, pl.ds(16 * j, 16)]
                    if nj > 1:
                        rep = _dgather(ex, jnp.where(lane < 8, 2 * j, 2 * j + 1))
                    else:
                        rep = ex
                    msg_v[e, pl.ds(16 * j, 16)] = hvec * rep
                msg_v[e, pl.ds(fw, 16)] = ex
                return c2

            lax.fori_loop(0, BLK, edge_body, 0)
            pltpu.sync_copy(msg_v, accum.at[didx], add=True)

        pltpu.sync_copy(z_h.at[pl.ds(lo, ROWS_PT)], accum.at[pl.ds(lo, ROWS_PT)])
        plsc.subcore_barrier()

        fire(bufs[0], 0)

        def blk_body(i, carry):
            fire(bufs[1], 2 * i + 1)
            drain(bufs[0])
            compute_scatter(bufs[0])
            fire(bufs[0], 2 * i + 2)
            drain(bufs[1])
            compute_scatter(bufs[1])
            return carry

        lax.fori_loop(0, EPT // BLK // 2, blk_body, 0)
        drain(bufs[0])

        plsc.subcore_barrier()
        pltpu.sync_copy(accum.at[pl.ds(lo, ROWS_PT)],
                        out_h.at[cid, pl.ds(lo, ROWS_PT)])

    return edge_kernel


_edge_pass_l1 = _make_edge_pass(F1, AW1)
_edge_pass_l2 = _make_edge_pass(C2, AW2)


def _mid_body(p_ref, r_ref, b1_ref, w2_ref, as2_ref, ad2_ref,
              h2_ref, asr_ref, adr_ref):
    acc = p_ref[0] + p_ref[1]
    num = acc[:, :F1]
    den = acc[:, F1:F1 + H1]
    invr = jnp.dot(1.0 / (den + 1e-16), r_ref[...],
                   preferred_element_type=jnp.float32)
    hl = num * invr + b1_ref[...]
    hf = jnp.where(hl > 0, hl, jnp.exp(hl) - 1.0)
    h2 = jnp.dot(hf, w2_ref[...], preferred_element_type=jnp.float32)
    h2_ref[...] = h2
    asr_ref[...] = jnp.dot(h2, as2_ref[...], preferred_element_type=jnp.float32)
    adr_ref[...] = jnp.dot(h2, ad2_ref[...], preferred_element_type=jnp.float32)


def _stage3(part1, R8, b1_2d, W2, AS2, AD2):
    blk = NP // 8
    return pl.pallas_call(
        _mid_body,
        grid=(8,),
        in_specs=[
            pl.BlockSpec((2, blk, AW1), lambda i: (0, i, 0)),
            pl.BlockSpec((H1, F1), lambda i: (0, 0)),
            pl.BlockSpec((1, F1), lambda i: (0, 0)),
            pl.BlockSpec((F1, C2), lambda i: (0, 0)),
            pl.BlockSpec((C2, 16), lambda i: (0, 0)),
            pl.BlockSpec((C2, 16), lambda i: (0, 0)),
        ],
        out_specs=[
            pl.BlockSpec((blk, C2), lambda i: (i, 0)),
            pl.BlockSpec((blk, 16), lambda i: (i, 0)),
            pl.BlockSpec((blk, 16), lambda i: (i, 0)),
        ],
        out_shape=[
            jax.ShapeDtypeStruct((NP, C2), jnp.float32),
            jax.ShapeDtypeStruct((NP, 16), jnp.float32),
            jax.ShapeDtypeStruct((NP, 16), jnp.float32),
        ],
    )(part1, R8, b1_2d, W2, AS2, AD2)


def _fin_body(p_ref, b2_ref, out_ref):
    acc = p_ref[0] + p_ref[1]
    o = acc[:, :C2] / (acc[:, C2:2 * C2] + 1e-16) + b2_ref[...]
    m = jnp.max(o, axis=1, keepdims=True)
    out_ref[...] = o - m - jnp.log(jnp.sum(jnp.exp(o - m), axis=1,
                                           keepdims=True))


def _stage5(part2, b2_2d):
    blk = NP // 8
    return pl.pallas_call(
        _fin_body,
        grid=(8,),
        in_specs=[
            pl.BlockSpec((2, blk, AW2), lambda i: (0, i, 0)),
            pl.BlockSpec((1, C2), lambda i: (0, 0)),
        ],
        out_specs=pl.BlockSpec((blk, C2), lambda i: (i, 0)),
        out_shape=jax.ShapeDtypeStruct((NP, C2), jnp.float32),
    )(part2, b2_2d)


def kernel(x, edge_index, W1, a_s1, a_d1, b1, W2, a_s2, a_d2, b2):
    f32 = jnp.float32
    x_pad = jnp.pad(x, ((0, NP - N), (0, 0)))

    loops = jnp.arange(N, dtype=jnp.int32)
    pad = jnp.full((EALLOC - E - N,), N, dtype=jnp.int32)
    src = jnp.concatenate([edge_index[0].astype(jnp.int32), loops, pad])
    dst = jnp.concatenate([edge_index[1].astype(jnp.int32), loops, pad])

    rows = jnp.arange(F1)
    AS1 = jnp.zeros((F1, 16), f32).at[rows, rows // C1].set(a_s1.reshape(F1))
    AD1 = jnp.zeros((F1, 16), f32).at[rows, rows // C1].set(a_d1.reshape(F1))

    h1, asrc1, adst1 = _stage1(x_pad, W1, AS1, AD1)

    zeros1 = jnp.zeros((NP, AW1), f32)
    part1 = _edge_pass_l1(src, dst, asrc1, adst1, h1, zeros1)

    R8 = jnp.zeros((H1, F1), f32).at[jnp.arange(F1) // C1, jnp.arange(F1)].set(1.0)
    AS2 = jnp.broadcast_to(a_s2.reshape(C2, 1), (C2, 16)).astype(f32)
    AD2 = jnp.broadcast_to(a_d2.reshape(C2, 1), (C2, 16)).astype(f32)
    h2, asrc2, adst2 = _stage3(part1, R8, b1.reshape(1, F1), W2, AS2, AD2)

    zeros2 = jnp.zeros((NP, AW2), f32)
    part2 = _edge_pass_l2(src, dst, asrc2, adst2, h2, zeros2)

    out = _stage5(part2, b2.reshape(1, C2))
    return out[:N]


# speedup vs baseline: 64.3554x; 1.0102x over previous
import functools

import jax
import jax.numpy as jnp
from jax import lax
from jax.experimental import pallas as pl
from jax.experimental.pallas import tpu as pltpu
from jax.experimental.pallas import tpu_sc as plsc

N = 10000
NP = 10240
E = 320000
EPP = 335872
EPT = EPP // 32
BLK = 128
EALLOC = EPP + BLK
F_IN = 128
H1 = 8
C1 = 8
F1 = H1 * C1
C2 = 16
AW1 = 80
AW2 = 32
ROWS_PT = NP // 16


def _mm1_body(x_ref, w_ref, as_ref, ad_ref, h_ref, asrc_ref, adst_ref):
    h = jnp.dot(x_ref[...], w_ref[...], preferred_element_type=jnp.float32)
    h_ref[...] = h
    asrc_ref[...] = jnp.dot(h, as_ref[...], preferred_element_type=jnp.float32)
    adst_ref[...] = jnp.dot(h, ad_ref[...], preferred_element_type=jnp.float32)


def _stage1(x_pad, W1, AS1, AD1):
    blk = NP // 8
    return pl.pallas_call(
        _mm1_body,
        grid=(8,),
        in_specs=[
            pl.BlockSpec((blk, F_IN), lambda i: (i, 0)),
            pl.BlockSpec((F_IN, F1), lambda i: (0, 0)),
            pl.BlockSpec((F1, 16), lambda i: (0, 0)),
            pl.BlockSpec((F1, 16), lambda i: (0, 0)),
        ],
        out_specs=[
            pl.BlockSpec((blk, F1), lambda i: (i, 0)),
            pl.BlockSpec((blk, 16), lambda i: (i, 0)),
            pl.BlockSpec((blk, 16), lambda i: (i, 0)),
        ],
        out_shape=[
            jax.ShapeDtypeStruct((NP, F1), jnp.float32),
            jax.ShapeDtypeStruct((NP, 16), jnp.float32),
            jax.ShapeDtypeStruct((NP, 16), jnp.float32),
        ],
    )(x_pad, W1, AS1, AD1)


def _dgather(v, idx):
    return lax.gather(
        v, idx[:, None],
        lax.GatherDimensionNumbers(
            offset_dims=(), collapsed_slice_dims=(0,), start_index_map=(0,)),
        slice_sizes=(1,),
        mode=lax.GatherScatterMode.PROMISE_IN_BOUNDS)


def _make_edge_pass(fw, aw):
    nj = fw // 16
    mesh = plsc.VectorSubcoreMesh(core_axis_name="c", subcore_axis_name="s")

    buf_types = []
    for _ in range(2):
        buf_types += [
            pltpu.VMEM((BLK,), jnp.int32),
            pltpu.VMEM((BLK,), jnp.int32),
            pltpu.VMEM((BLK, 16), jnp.float32),
            pltpu.VMEM((BLK, 16), jnp.float32),
            pltpu.VMEM((BLK, fw), jnp.float32),
            pltpu.VMEM((BLK, aw), jnp.float32),
            pltpu.SemaphoreType.DMA,
        ]

    @functools.partial(
        pl.kernel,
        mesh=mesh,
        compiler_params=pltpu.CompilerParams(use_tc_tiling_on_sc=False),
        out_type=jax.ShapeDtypeStruct((2, NP, aw), jnp.float32),
        scratch_types=buf_types + [pltpu.VMEM_SHARED((NP, aw), jnp.float32)],
    )
    def edge_kernel(src_h, dst_h, as_h, ad_h, h_h, out_h, *scr):
        bufs = (scr[0:7], scr[7:14])
        accum = scr[14]
        cid = lax.axis_index("c")
        sid = lax.axis_index("s")
        wid = cid * 16 + sid
        lo = sid * ROWS_PT

        def fire(buf, g):
            sidx, didx, as_v, ad_v, h_v, _, sem = buf
            base = wid * EPT + g * BLK
            pltpu.sync_copy(src_h.at[pl.ds(base, BLK)], sidx)
            pltpu.sync_copy(dst_h.at[pl.ds(base, BLK)], didx)
            pltpu.async_copy(as_h.at[sidx], as_v, sem)
            pltpu.async_copy(ad_h.at[didx], ad_v, sem)
            pltpu.async_copy(h_h.at[sidx], h_v, sem)

        def drain(buf):
            sidx, didx, as_v, ad_v, h_v, _, sem = buf
            pltpu.make_async_copy(as_h.at[sidx], as_v, sem).wait()
            pltpu.make_async_copy(ad_h.at[didx], ad_v, sem).wait()
            pltpu.make_async_copy(h_h.at[sidx], h_v, sem).wait()

        def compute_scatter(buf):
            _, didx, as_v, ad_v, h_v, msg_v, _ = buf
            lane = lax.iota(jnp.int32, 16)

            def do_edge(e):
                ea = as_v[e] + ad_v[e]
                ea = jnp.where(ea >= 0, ea, 0.2 * ea)
                ex = jnp.exp(ea)
                for j in range(nj):
                    hvec = h_v[e, pl.ds(16 * j, 16)]
                    if nj > 1:
                        rep = _dgather(ex, jnp.where(lane < 8, 2 * j, 2 * j + 1))
                    else:
                        rep = ex
                    msg_v[e, pl.ds(16 * j, 16)] = hvec * rep
                msg_v[e, pl.ds(fw, 16)] = ex

            def edge_body(k, c2):
                do_edge(2 * k)
                do_edge(2 * k + 1)
                return c2

            lax.fori_loop(0, BLK // 2, edge_body, 0)
            pltpu.sync_copy(msg_v, accum.at[didx], add=True)

        fire(bufs[0], 0)

        zmsg = bufs[0][5]
        zvec = jnp.zeros((16,), jnp.float32)

        def zrow(r, c2):
            for j in range(aw // 16):
                zmsg[r, pl.ds(16 * j, 16)] = zvec
            return c2

        lax.fori_loop(0, BLK, zrow, 0)
        for k in range(ROWS_PT // BLK):
            pltpu.sync_copy(zmsg, accum.at[pl.ds(lo + k * BLK, BLK)])
        plsc.subcore_barrier()

        def blk_body(i, carry):
            fire(bufs[1], 2 * i + 1)
            drain(bufs[0])
            compute_scatter(bufs[0])
            fire(bufs[0], 2 * i + 2)
            drain(bufs[1])
            compute_scatter(bufs[1])
            return carry

        lax.fori_loop(0, EPT // BLK // 2, blk_body, 0)
        drain(bufs[0])

        plsc.subcore_barrier()
        pltpu.sync_copy(accum.at[pl.ds(lo, ROWS_PT)],
                        out_h.at[cid, pl.ds(lo, ROWS_PT)])

    return edge_kernel


_edge_pass_l1 = _make_edge_pass(F1, AW1)
_edge_pass_l2 = _make_edge_pass(C2, AW2)


def _mid_body(p_ref, r_ref, b1_ref, w2_ref, as2_ref, ad2_ref,
              h2_ref, asr_ref, adr_ref):
    acc = p_ref[0] + p_ref[1]
    num = acc[:, :F1]
    den = acc[:, F1:F1 + H1]
    invr = jnp.dot(1.0 / (den + 1e-16), r_ref[...],
                   preferred_element_type=jnp.float32)
    hl = num * invr + b1_ref[...]
    hf = jnp.where(hl > 0, hl, jnp.exp(hl) - 1.0)
    h2 = jnp.dot(hf, w2_ref[...], preferred_element_type=jnp.float32)
    h2_ref[...] = h2
    asr_ref[...] = jnp.dot(h2, as2_ref[...], preferred_element_type=jnp.float32)
    adr_ref[...] = jnp.dot(h2, ad2_ref[...], preferred_element_type=jnp.float32)


def _stage3(part1, R8, b1_2d, W2, AS2, AD2):
    blk = NP // 8
    return pl.pallas_call(
        _mid_body,
        grid=(8,),
        in_specs=[
            pl.BlockSpec((2, blk, AW1), lambda i: (0, i, 0)),
            pl.BlockSpec((H1, F1), lambda i: (0, 0)),
            pl.BlockSpec((1, F1), lambda i: (0, 0)),
            pl.BlockSpec((F1, C2), lambda i: (0, 0)),
            pl.BlockSpec((C2, 16), lambda i: (0, 0)),
            pl.BlockSpec((C2, 16), lambda i: (0, 0)),
        ],
        out_specs=[
            pl.BlockSpec((blk, C2), lambda i: (i, 0)),
            pl.BlockSpec((blk, 16), lambda i: (i, 0)),
            pl.BlockSpec((blk, 16), lambda i: (i, 0)),
        ],
        out_shape=[
            jax.ShapeDtypeStruct((NP, C2), jnp.float32),
            jax.ShapeDtypeStruct((NP, 16), jnp.float32),
            jax.ShapeDtypeStruct((NP, 16), jnp.float32),
        ],
    )(part1, R8, b1_2d, W2, AS2, AD2)


def _fin_body(p_ref, b2_ref, out_ref):
    acc = p_ref[0] + p_ref[1]
    o = acc[:, :C2] / (acc[:, C2:2 * C2] + 1e-16) + b2_ref[...]
    m = jnp.max(o, axis=1, keepdims=True)
    out_ref[...] = o - m - jnp.log(jnp.sum(jnp.exp(o - m), axis=1,
                                           keepdims=True))


def _stage5(part2, b2_2d):
    blk = NP // 8
    return pl.pallas_call(
        _fin_body,
        grid=(8,),
        in_specs=[
            pl.BlockSpec((2, blk, AW2), lambda i: (0, i, 0)),
            pl.BlockSpec((1, C2), lambda i: (0, 0)),
        ],
        out_specs=pl.BlockSpec((blk, C2), lambda i: (i, 0)),
        out_shape=jax.ShapeDtypeStruct((NP, C2), jnp.float32),
    )(part2, b2_2d)


def kernel(x, edge_index, W1, a_s1, a_d1, b1, W2, a_s2, a_d2, b2):
    f32 = jnp.float32
    x_pad = jnp.pad(x, ((0, NP - N), (0, 0)))

    loops = jnp.arange(N, dtype=jnp.int32)
    pad = jnp.full((EALLOC - E - N,), N, dtype=jnp.int32)
    src = jnp.concatenate([edge_index[0].astype(jnp.int32), loops, pad])
    dst = jnp.concatenate([edge_index[1].astype(jnp.int32), loops, pad])

    rows = jnp.arange(F1)
    AS1 = jnp.zeros((F1, 16), f32).at[rows, rows // C1].set(a_s1.reshape(F1))
    AD1 = jnp.zeros((F1, 16), f32).at[rows, rows // C1].set(a_d1.reshape(F1))

    h1, asrc1, adst1 = _stage1(x_pad, W1, AS1, AD1)

    part1 = _edge_pass_l1(src, dst, asrc1, adst1, h1)

    R8 = jnp.zeros((H1, F1), f32).at[jnp.arange(F1) // C1, jnp.arange(F1)].set(1.0)
    AS2 = jnp.broadcast_to(a_s2.reshape(C2, 1), (C2, 16)).astype(f32)
    AD2 = jnp.broadcast_to(a_d2.reshape(C2, 1), (C2, 16)).astype(f32)
    h2, asrc2, adst2 = _stage3(part1, R8, b1.reshape(1, F1), W2, AS2, AD2)

    part2 = _edge_pass_l2(src, dst, asrc2, adst2, h2)

    out = _stage5(part2, b2.reshape(1, C2))
    return out[:N]


# speedup vs baseline: 68.2960x; 1.0612x over previous
import functools

import jax
import jax.numpy as jnp
from jax import lax
from jax.experimental import pallas as pl
from jax.experimental.pallas import tpu as pltpu
from jax.experimental.pallas import tpu_sc as plsc

N = 10000
NP = 10240
E = 320000
EPP = 335872
EPT = EPP // 32
BLK = 128
EALLOC = EPP + BLK
F_IN = 128
H1 = 8
C1 = 8
F1 = H1 * C1
C2 = 16
AW1 = 80
AW2 = 32
ROWS_PT = NP // 16


def _mm1_body(x_ref, w_ref, as_ref, ad_ref, h_ref, asrc_ref, adst_ref):
    h = jnp.dot(x_ref[...], w_ref[...], preferred_element_type=jnp.float32)
    h_ref[...] = h
    asrc_ref[...] = jnp.dot(h, as_ref[...], preferred_element_type=jnp.float32)
    adst_ref[...] = jnp.dot(h, ad_ref[...], preferred_element_type=jnp.float32)


def _stage1(x_pad, W1, AS1, AD1):
    blk = NP // 8
    return pl.pallas_call(
        _mm1_body,
        grid=(8,),
        in_specs=[
            pl.BlockSpec((blk, F_IN), lambda i: (i, 0)),
            pl.BlockSpec((F_IN, F1), lambda i: (0, 0)),
            pl.BlockSpec((F1, 16), lambda i: (0, 0)),
            pl.BlockSpec((F1, 16), lambda i: (0, 0)),
        ],
        out_specs=[
            pl.BlockSpec((blk, F1), lambda i: (i, 0)),
            pl.BlockSpec((blk, 16), lambda i: (i, 0)),
            pl.BlockSpec((blk, 16), lambda i: (i, 0)),
        ],
        out_shape=[
            jax.ShapeDtypeStruct((NP, F1), jnp.float32),
            jax.ShapeDtypeStruct((NP, 16), jnp.float32),
            jax.ShapeDtypeStruct((NP, 16), jnp.float32),
        ],
    )(x_pad, W1, AS1, AD1)


def _dgather(v, idx):
    return lax.gather(
        v, idx[:, None],
        lax.GatherDimensionNumbers(
            offset_dims=(), collapsed_slice_dims=(0,), start_index_map=(0,)),
        slice_sizes=(1,),
        mode=lax.GatherScatterMode.PROMISE_IN_BOUNDS)


def _make_edge_pass(fw, aw):
    nj = fw // 16
    mesh = plsc.VectorSubcoreMesh(core_axis_name="c", subcore_axis_name="s")

    buf_types = []
    for _ in range(2):
        buf_types += [
            pltpu.VMEM((BLK,), jnp.int32),
            pltpu.VMEM((BLK,), jnp.int32),
            pltpu.VMEM((BLK, 16), jnp.float32),
            pltpu.VMEM((BLK, 16), jnp.float32),
            pltpu.VMEM((BLK, fw), jnp.float32),
            pltpu.VMEM((BLK, aw), jnp.float32),
            pltpu.VMEM((BLK,), jnp.int32),
            pltpu.SemaphoreType.DMA,
            pltpu.SemaphoreType.DMA,
        ]

    @functools.partial(
        pl.kernel,
        mesh=mesh,
        compiler_params=pltpu.CompilerParams(use_tc_tiling_on_sc=False),
        out_type=jax.ShapeDtypeStruct((2, NP, aw), jnp.float32),
        scratch_types=buf_types + [pltpu.VMEM_SHARED((NP, aw), jnp.float32)],
    )
    def edge_kernel(src_h, dst_h, as_h, ad_h, h_h, out_h, *scr):
        bufs = (scr[0:9], scr[9:18])
        accum = scr[18]
        cid = lax.axis_index("c")
        sid = lax.axis_index("s")
        wid = cid * 16 + sid
        lo = sid * ROWS_PT

        def fire(buf, g):
            sidx, didx, as_v, ad_v, h_v, _, _, sem, _ = buf
            base = wid * EPT + g * BLK
            pltpu.sync_copy(src_h.at[pl.ds(base, BLK)], sidx)
            pltpu.sync_copy(dst_h.at[pl.ds(base, BLK)], didx)
            pltpu.async_copy(as_h.at[sidx], as_v, sem)
            pltpu.async_copy(ad_h.at[didx], ad_v, sem)
            pltpu.async_copy(h_h.at[sidx], h_v, sem)

        def drain(buf):
            sidx, didx, as_v, ad_v, h_v, _, _, sem, _ = buf
            pltpu.make_async_copy(as_h.at[sidx], as_v, sem).wait()
            pltpu.make_async_copy(ad_h.at[didx], ad_v, sem).wait()
            pltpu.make_async_copy(h_h.at[sidx], h_v, sem).wait()

        def drain_scatter(buf):
            _, _, _, _, _, msg_v, didx_s, _, ssem = buf
            pltpu.make_async_copy(msg_v, accum.at[didx_s], ssem).wait()

        def compute_scatter(buf):
            _, didx, as_v, ad_v, h_v, msg_v, didx_s, _, ssem = buf
            lane = lax.iota(jnp.int32, 16)

            def do_edge(e):
                ea = as_v[e] + ad_v[e]
                ea = jnp.where(ea >= 0, ea, 0.2 * ea)
                ex = jnp.exp(ea)
                for j in range(nj):
                    hvec = h_v[e, pl.ds(16 * j, 16)]
                    if nj > 1:
                        rep = _dgather(ex, jnp.where(lane < 8, 2 * j, 2 * j + 1))
                    else:
                        rep = ex
                    msg_v[e, pl.ds(16 * j, 16)] = hvec * rep
                msg_v[e, pl.ds(fw, 16)] = ex

            def edge_body(k, c2):
                do_edge(2 * k)
                do_edge(2 * k + 1)
                return c2

            lax.fori_loop(0, BLK // 2, edge_body, 0)
            for t in range(BLK // 16):
                didx_s[pl.ds(16 * t, 16)] = didx[pl.ds(16 * t, 16)]
            pltpu.async_copy(msg_v, accum.at[didx_s], ssem, add=True)

        fire(bufs[0], 0)

        zmsg = bufs[0][5]
        zvec = jnp.zeros((16,), jnp.float32)

        def zrow(r, c2):
            for j in range(aw // 16):
                zmsg[r, pl.ds(16 * j, 16)] = zvec
            return c2

        lax.fori_loop(0, BLK, zrow, 0)
        for k in range(ROWS_PT // BLK):
            pltpu.sync_copy(zmsg, accum.at[pl.ds(lo + k * BLK, BLK)])
        plsc.subcore_barrier()

        fire(bufs[1], 1)
        drain(bufs[0])
        compute_scatter(bufs[0])
        fire(bufs[0], 2)
        drain(bufs[1])
        compute_scatter(bufs[1])

        def blk_body(i, carry):
            fire(bufs[1], 2 * i + 1)
            drain(bufs[0])
            drain_scatter(bufs[0])
            compute_scatter(bufs[0])
            fire(bufs[0], 2 * i + 2)
            drain(bufs[1])
            drain_scatter(bufs[1])
            compute_scatter(bufs[1])
            return carry

        lax.fori_loop(1, EPT // BLK // 2, blk_body, 0)
        drain(bufs[0])
        drain_scatter(bufs[0])
        drain_scatter(bufs[1])

        plsc.subcore_barrier()
        pltpu.sync_copy(accum.at[pl.ds(lo, ROWS_PT)],
                        out_h.at[cid, pl.ds(lo, ROWS_PT)])

    return edge_kernel


_edge_pass_l1 = _make_edge_pass(F1, AW1)
_edge_pass_l2 = _make_edge_pass(C2, AW2)


def _mid_body(p_ref, r_ref, b1_ref, w2_ref, as2_ref, ad2_ref,
              h2_ref, asr_ref, adr_ref):
    acc = p_ref[0] + p_ref[1]
    num = acc[:, :F1]
    den = acc[:, F1:F1 + H1]
    invr = jnp.dot(1.0 / (den + 1e-16), r_ref[...],
                   preferred_element_type=jnp.float32)
    hl = num * invr + b1_ref[...]
    hf = jnp.where(hl > 0, hl, jnp.exp(hl) - 1.0)
    h2 = jnp.dot(hf, w2_ref[...], preferred_element_type=jnp.float32)
    h2_ref[...] = h2
    asr_ref[...] = jnp.dot(h2, as2_ref[...], preferred_element_type=jnp.float32)
    adr_ref[...] = jnp.dot(h2, ad2_ref[...], preferred_element_type=jnp.float32)


def _stage3(part1, R8, b1_2d, W2, AS2, AD2):
    blk = NP // 8
    return pl.pallas_call(
        _mid_body,
        grid=(8,),
        in_specs=[
            pl.BlockSpec((2, blk, AW1), lambda i: (0, i, 0)),
            pl.BlockSpec((H1, F1), lambda i: (0, 0)),
            pl.BlockSpec((1, F1), lambda i: (0, 0)),
            pl.BlockSpec((F1, C2), lambda i: (0, 0)),
            pl.BlockSpec((C2, 16), lambda i: (0, 0)),
            pl.BlockSpec((C2, 16), lambda i: (0, 0)),
        ],
        out_specs=[
            pl.BlockSpec((blk, C2), lambda i: (i, 0)),
            pl.BlockSpec((blk, 16), lambda i: (i, 0)),
            pl.BlockSpec((blk, 16), lambda i: (i, 0)),
        ],
        out_shape=[
            jax.ShapeDtypeStruct((NP, C2), jnp.float32),
            jax.ShapeDtypeStruct((NP, 16), jnp.float32),
            jax.ShapeDtypeStruct((NP, 16), jnp.float32),
        ],
    )(part1, R8, b1_2d, W2, AS2, AD2)


def _fin_body(p_ref, b2_ref, out_ref):
    acc = p_ref[0] + p_ref[1]
    o = acc[:, :C2] / (acc[:, C2:2 * C2] + 1e-16) + b2_ref[...]
    m = jnp.max(o, axis=1, keepdims=True)
    out_ref[...] = o - m - jnp.log(jnp.sum(jnp.exp(o - m), axis=1,
                                           keepdims=True))


def _stage5(part2, b2_2d):
    blk = NP // 8
    return pl.pallas_call(
        _fin_body,
        grid=(8,),
        in_specs=[
            pl.BlockSpec((2, blk, AW2), lambda i: (0, i, 0)),
            pl.BlockSpec((1, C2), lambda i: (0, 0)),
        ],
        out_specs=pl.BlockSpec((blk, C2), lambda i: (i, 0)),
        out_shape=jax.ShapeDtypeStruct((NP, C2), jnp.float32),
    )(part2, b2_2d)


def kernel(x, edge_index, W1, a_s1, a_d1, b1, W2, a_s2, a_d2, b2):
    f32 = jnp.float32
    x_pad = jnp.pad(x, ((0, NP - N), (0, 0)))

    loops = jnp.arange(N, dtype=jnp.int32)
    pad = jnp.full((EALLOC - E - N,), N, dtype=jnp.int32)
    src = jnp.concatenate([edge_index[0].astype(jnp.int32), loops, pad])
    dst = jnp.concatenate([edge_index[1].astype(jnp.int32), loops, pad])

    rows = jnp.arange(F1)
    AS1 = jnp.zeros((F1, 16), f32).at[rows, rows // C1].set(a_s1.reshape(F1))
    AD1 = jnp.zeros((F1, 16), f32).at[rows, rows // C1].set(a_d1.reshape(F1))

    h1, asrc1, adst1 = _stage1(x_pad, W1, AS1, AD1)

    part1 = _edge_pass_l1(src, dst, asrc1, adst1, h1)

    R8 = jnp.zeros((H1, F1), f32).at[jnp.arange(F1) // C1, jnp.arange(F1)].set(1.0)
    AS2 = jnp.broadcast_to(a_s2.reshape(C2, 1), (C2, 16)).astype(f32)
    AD2 = jnp.broadcast_to(a_d2.reshape(C2, 1), (C2, 16)).astype(f32)
    h2, asrc2, adst2 = _stage3(part1, R8, b1.reshape(1, F1), W2, AS2, AD2)

    part2 = _edge_pass_l2(src, dst, asrc2, adst2, h2)

    out = _stage5(part2, b2.reshape(1, C2))
    return out[:N]


# speedup vs baseline: 74.8579x; 1.0961x over previous
import functools

import jax
import jax.numpy as jnp
from jax import lax
from jax.experimental import pallas as pl
from jax.experimental.pallas import tpu as pltpu
from jax.experimental.pallas import tpu_sc as plsc

N = 10000
NP = 10240
E = 320000
EPP = 335872
EPT = EPP // 32
BLK = 128
EALLOC = EPP + BLK
F_IN = 128
H1 = 8
C1 = 8
F1 = H1 * C1
C2 = 16
AW1 = 80
AW2 = 32
ROWS_PT = NP // 16


def _mm1_body(x_ref, w_ref, as_ref, ad_ref, h_ref, asrc_ref, adst_ref):
    h = jnp.dot(x_ref[...], w_ref[...], preferred_element_type=jnp.float32)
    h_ref[...] = h
    asrc_ref[...] = jnp.dot(h, as_ref[...], preferred_element_type=jnp.float32)
    adst_ref[...] = jnp.dot(h, ad_ref[...], preferred_element_type=jnp.float32)


def _stage1(x_pad, W1, AS1, AD1):
    blk = NP // 8
    return pl.pallas_call(
        _mm1_body,
        grid=(8,),
        in_specs=[
            pl.BlockSpec((blk, F_IN), lambda i: (i, 0)),
            pl.BlockSpec((F_IN, F1), lambda i: (0, 0)),
            pl.BlockSpec((F1, 16), lambda i: (0, 0)),
            pl.BlockSpec((F1, 16), lambda i: (0, 0)),
        ],
        out_specs=[
            pl.BlockSpec((blk, F1), lambda i: (i, 0)),
            pl.BlockSpec((blk, 16), lambda i: (i, 0)),
            pl.BlockSpec((blk, 16), lambda i: (i, 0)),
        ],
        out_shape=[
            jax.ShapeDtypeStruct((NP, F1), jnp.float32),
            jax.ShapeDtypeStruct((NP, 16), jnp.float32),
            jax.ShapeDtypeStruct((NP, 16), jnp.float32),
        ],
    )(x_pad, W1, AS1, AD1)


def _dgather(v, idx):
    return lax.gather(
        v, idx[:, None],
        lax.GatherDimensionNumbers(
            offset_dims=(), collapsed_slice_dims=(0,), start_index_map=(0,)),
        slice_sizes=(1,),
        mode=lax.GatherScatterMode.PROMISE_IN_BOUNDS)


def _make_edge_pass(fw, aw):
    nj = fw // 16
    mesh = plsc.VectorSubcoreMesh(core_axis_name="c", subcore_axis_name="s")

    buf_types = []
    for _ in range(2):
        buf_types += [
            pltpu.VMEM((2, BLK), jnp.int32),
            pltpu.VMEM((BLK, 16), jnp.float32),
            pltpu.VMEM((BLK, 16), jnp.float32),
            pltpu.VMEM((BLK, fw), jnp.float32),
            pltpu.VMEM((BLK, aw), jnp.float32),
            pltpu.VMEM((BLK,), jnp.int32),
            pltpu.SemaphoreType.DMA,
            pltpu.SemaphoreType.DMA,
        ]

    @functools.partial(
        pl.kernel,
        mesh=mesh,
        compiler_params=pltpu.CompilerParams(use_tc_tiling_on_sc=False),
        out_type=jax.ShapeDtypeStruct((2, NP, aw), jnp.float32),
        scratch_types=buf_types + [pltpu.VMEM_SHARED((NP, aw), jnp.float32)],
    )
    def edge_kernel(sd_h, as_h, ad_h, h_h, out_h, *scr):
        bufs = (scr[0:8], scr[8:16])
        accum = scr[16]
        cid = lax.axis_index("c")
        sid = lax.axis_index("s")
        wid = cid * 16 + sid
        lo = sid * ROWS_PT
        nbt = EPT // BLK

        def fire(buf, g):
            sdidx, as_v, ad_v, h_v, _, _, sem, _ = buf
            pltpu.sync_copy(sd_h.at[pl.ds((wid * nbt + g) * 2, 2)], sdidx)
            pltpu.async_copy(as_h.at[sdidx.at[0]], as_v, sem)
            pltpu.async_copy(ad_h.at[sdidx.at[1]], ad_v, sem)
            pltpu.async_copy(h_h.at[sdidx.at[0]], h_v, sem)

        def drain(buf):
            sdidx, as_v, ad_v, h_v, _, _, sem, _ = buf
            pltpu.make_async_copy(as_h.at[sdidx.at[0]], as_v, sem).wait()
            pltpu.make_async_copy(ad_h.at[sdidx.at[1]], ad_v, sem).wait()
            pltpu.make_async_copy(h_h.at[sdidx.at[0]], h_v, sem).wait()

        def drain_scatter(buf):
            _, _, _, _, msg_v, didx_s, _, ssem = buf
            pltpu.make_async_copy(msg_v, accum.at[didx_s], ssem).wait()

        def compute_scatter(buf):
            sdidx, as_v, ad_v, h_v, msg_v, didx_s, _, ssem = buf
            lane = lax.iota(jnp.int32, 16)
            reps = [jnp.where(lane < 8, 2 * j, 2 * j + 1) for j in range(nj)]

            def do_edge(e):
                ea = as_v[e] + ad_v[e]
                ea = jnp.where(ea >= 0, ea, 0.2 * ea)
                ex = jnp.exp(ea)
                for j in range(nj):
                    hvec = h_v[e, pl.ds(16 * j, 16)]
                    rep = _dgather(ex, reps[j]) if nj > 1 else ex
                    msg_v[e, pl.ds(16 * j, 16)] = hvec * rep
                msg_v[e, pl.ds(fw, 16)] = ex

            def edge_body(k, c2):
                do_edge(2 * k)
                do_edge(2 * k + 1)
                return c2

            lax.fori_loop(0, BLK // 2, edge_body, 0)
            for t in range(BLK // 16):
                didx_s[pl.ds(16 * t, 16)] = sdidx[1, pl.ds(16 * t, 16)]
            pltpu.async_copy(msg_v, accum.at[didx_s], ssem, add=True)

        fire(bufs[0], 0)

        zmsg = bufs[0][4]
        zvec = jnp.zeros((16,), jnp.float32)

        def zrow(r, c2):
            for j in range(aw // 16):
                zmsg[r, pl.ds(16 * j, 16)] = zvec
            return c2

        lax.fori_loop(0, BLK, zrow, 0)
        for k in range(ROWS_PT // BLK):
            pltpu.sync_copy(zmsg, accum.at[pl.ds(lo + k * BLK, BLK)])
        plsc.subcore_barrier()

        fire(bufs[1], 1)
        drain(bufs[0])
        compute_scatter(bufs[0])
        fire(bufs[0], 2)
        drain(bufs[1])
        compute_scatter(bufs[1])

        def blk_body(i, carry):
            fire(bufs[1], 2 * i + 1)
            drain(bufs[0])
            drain_scatter(bufs[0])
            compute_scatter(bufs[0])
            fire(bufs[0], 2 * i + 2)
            drain(bufs[1])
            drain_scatter(bufs[1])
            compute_scatter(bufs[1])
            return carry

        lax.fori_loop(1, EPT // BLK // 2, blk_body, 0)
        drain(bufs[0])
        drain_scatter(bufs[0])
        drain_scatter(bufs[1])

        plsc.subcore_barrier()
        pltpu.sync_copy(accum.at[pl.ds(lo, ROWS_PT)],
                        out_h.at[cid, pl.ds(lo, ROWS_PT)])

    return edge_kernel


_edge_pass_l1 = _make_edge_pass(F1, AW1)
_edge_pass_l2 = _make_edge_pass(C2, AW2)


def _mid_body(p_ref, r_ref, b1_ref, w2_ref, as2_ref, ad2_ref,
              h2_ref, asr_ref, adr_ref):
    acc = p_ref[0] + p_ref[1]
    num = acc[:, :F1]
    den = acc[:, F1:F1 + H1]
    invr = jnp.dot(1.0 / (den + 1e-16), r_ref[...],
                   preferred_element_type=jnp.float32)
    hl = num * invr + b1_ref[...]
    hf = jnp.where(hl > 0, hl, jnp.exp(hl) - 1.0)
    h2 = jnp.dot(hf, w2_ref[...], preferred_element_type=jnp.float32)
    h2_ref[...] = h2
    asr_ref[...] = jnp.dot(h2, as2_ref[...], preferred_element_type=jnp.float32)
    adr_ref[...] = jnp.dot(h2, ad2_ref[...], preferred_element_type=jnp.float32)


def _stage3(part1, R8, b1_2d, W2, AS2, AD2):
    blk = NP // 8
    return pl.pallas_call(
        _mid_body,
        grid=(8,),
        in_specs=[
            pl.BlockSpec((2, blk, AW1), lambda i: (0, i, 0)),
            pl.BlockSpec((H1, F1), lambda i: (0, 0)),
            pl.BlockSpec((1, F1), lambda i: (0, 0)),
            pl.BlockSpec((F1, C2), lambda i: (0, 0)),
            pl.BlockSpec((C2, 16), lambda i: (0, 0)),
            pl.BlockSpec((C2, 16), lambda i: (0, 0)),
        ],
        out_specs=[
            pl.BlockSpec((blk, C2), lambda i: (i, 0)),
            pl.BlockSpec((blk, 16), lambda i: (i, 0)),
            pl.BlockSpec((blk, 16), lambda i: (i, 0)),
        ],
        out_shape=[
            jax.ShapeDtypeStruct((NP, C2), jnp.float32),
            jax.ShapeDtypeStruct((NP, 16), jnp.float32),
            jax.ShapeDtypeStruct((NP, 16), jnp.float32),
        ],
    )(part1, R8, b1_2d, W2, AS2, AD2)


def _fin_body(p_ref, b2_ref, out_ref):
    acc = p_ref[0] + p_ref[1]
    o = acc[:, :C2] / (acc[:, C2:2 * C2] + 1e-16) + b2_ref[...]
    m = jnp.max(o, axis=1, keepdims=True)
    out_ref[...] = o - m - jnp.log(jnp.sum(jnp.exp(o - m), axis=1,
                                           keepdims=True))


def _stage5(part2, b2_2d):
    blk = NP // 8
    return pl.pallas_call(
        _fin_body,
        grid=(8,),
        in_specs=[
            pl.BlockSpec((2, blk, AW2), lambda i: (0, i, 0)),
            pl.BlockSpec((1, C2), lambda i: (0, 0)),
        ],
        out_specs=pl.BlockSpec((blk, C2), lambda i: (i, 0)),
        out_shape=jax.ShapeDtypeStruct((NP, C2), jnp.float32),
    )(part2, b2_2d)


def kernel(x, edge_index, W1, a_s1, a_d1, b1, W2, a_s2, a_d2, b2):
    f32 = jnp.float32
    x_pad = jnp.pad(x, ((0, NP - N), (0, 0)))

    loops = jnp.arange(N, dtype=jnp.int32)
    pad = jnp.full((EALLOC - E - N,), N, dtype=jnp.int32)
    src = jnp.concatenate([edge_index[0].astype(jnp.int32), loops, pad])
    dst = jnp.concatenate([edge_index[1].astype(jnp.int32), loops, pad])
    sd = jnp.stack([src.reshape(-1, BLK), dst.reshape(-1, BLK)],
                   axis=1).reshape(-1, BLK)

    rows = jnp.arange(F1)
    AS1 = jnp.zeros((F1, 16), f32).at[rows, rows // C1].set(a_s1.reshape(F1))
    AD1 = jnp.zeros((F1, 16), f32).at[rows, rows // C1].set(a_d1.reshape(F1))

    h1, asrc1, adst1 = _stage1(x_pad, W1, AS1, AD1)

    part1 = _edge_pass_l1(sd, asrc1, adst1, h1)

    R8 = jnp.zeros((H1, F1), f32).at[jnp.arange(F1) // C1, jnp.arange(F1)].set(1.0)
    AS2 = jnp.broadcast_to(a_s2.reshape(C2, 1), (C2, 16)).astype(f32)
    AD2 = jnp.broadcast_to(a_d2.reshape(C2, 1), (C2, 16)).astype(f32)
    h2, asrc2, adst2 = _stage3(part1, R8, b1.reshape(1, F1), W2, AS2, AD2)

    part2 = _edge_pass_l2(sd, asrc2, adst2, h2)

    out = _stage5(part2, b2.reshape(1, C2))
    return out[:N]


# speedup vs baseline: 76.0786x; 1.0163x over previous
import functools

import jax
import jax.numpy as jnp
from jax import lax
from jax.experimental import pallas as pl
from jax.experimental.pallas import tpu as pltpu
from jax.experimental.pallas import tpu_sc as plsc

N = 10000
NP = 10240
E = 320000
EPP = 344064
EPT = EPP // 32
BLK = 128
SB = 256
EALLOC = EPP + SB
F_IN = 128
H1 = 8
C1 = 8
F1 = H1 * C1
C2 = 16
AW1 = 80
AW2 = 32
ROWS_PT = NP // 16


def _mm1_body(x_ref, w_ref, as_ref, ad_ref, h_ref, asrc_ref, adst_ref):
    h = jnp.dot(x_ref[...], w_ref[...], preferred_element_type=jnp.float32)
    h_ref[...] = h
    asrc_ref[...] = jnp.dot(h, as_ref[...], preferred_element_type=jnp.float32)
    adst_ref[...] = jnp.dot(h, ad_ref[...], preferred_element_type=jnp.float32)


def _stage1(x_pad, W1, AS1, AD1):
    blk = NP // 8
    return pl.pallas_call(
        _mm1_body,
        grid=(8,),
        in_specs=[
            pl.BlockSpec((blk, F_IN), lambda i: (i, 0)),
            pl.BlockSpec((F_IN, F1), lambda i: (0, 0)),
            pl.BlockSpec((F1, 16), lambda i: (0, 0)),
            pl.BlockSpec((F1, 16), lambda i: (0, 0)),
        ],
        out_specs=[
            pl.BlockSpec((blk, F1), lambda i: (i, 0)),
            pl.BlockSpec((blk, 16), lambda i: (i, 0)),
            pl.BlockSpec((blk, 16), lambda i: (i, 0)),
        ],
        out_shape=[
            jax.ShapeDtypeStruct((NP, F1), jnp.float32),
            jax.ShapeDtypeStruct((NP, 16), jnp.float32),
            jax.ShapeDtypeStruct((NP, 16), jnp.float32),
        ],
    )(x_pad, W1, AS1, AD1)


def _dgather(v, idx):
    return lax.gather(
        v, idx[:, None],
        lax.GatherDimensionNumbers(
            offset_dims=(), collapsed_slice_dims=(0,), start_index_map=(0,)),
        slice_sizes=(1,),
        mode=lax.GatherScatterMode.PROMISE_IN_BOUNDS)


def _make_edge_pass(fw, aw, ept, nh):
    nj = fw // 16
    mesh = plsc.VectorSubcoreMesh(core_axis_name="c", subcore_axis_name="s")

    nset = 4 * nh + 4
    buf_types = []
    for _ in range(2):
        buf_types += [pltpu.VMEM((2 * nh, BLK), jnp.int32)]
        buf_types += [pltpu.VMEM((BLK, 16), jnp.float32) for _ in range(nh)]
        buf_types += [pltpu.VMEM((BLK, 16), jnp.float32) for _ in range(nh)]
        buf_types += [pltpu.VMEM((BLK, fw), jnp.float32) for _ in range(nh)]
        buf_types += [pltpu.VMEM((BLK, aw), jnp.float32) for _ in range(nh)]
        buf_types += [
            pltpu.VMEM((nh, BLK), jnp.int32),
            pltpu.SemaphoreType.DMA,
            pltpu.SemaphoreType.DMA,
        ]

    @functools.partial(
        pl.kernel,
        mesh=mesh,
        compiler_params=pltpu.CompilerParams(use_tc_tiling_on_sc=False),
        out_type=jax.ShapeDtypeStruct((2, NP, aw), jnp.float32),
        scratch_types=buf_types + [pltpu.VMEM_SHARED((NP, aw), jnp.float32)],
    )
    def edge_kernel(sd_h, as_h, ad_h, h_h, out_h, *scr):
        bufs = (scr[0:nset], scr[nset:2 * nset])
        accum = scr[2 * nset]
        cid = lax.axis_index("c")
        sid = lax.axis_index("s")
        wid = cid * 16 + sid
        lo = sid * ROWS_PT
        nbt = ept // (nh * BLK)

        def _gathers(buf):
            sdidx, sem = buf[0], buf[4 * nh + 2]
            out = []
            for half in range(nh):
                si, di = sdidx.at[2 * half], sdidx.at[2 * half + 1]
                out.append((as_h.at[si], buf[1 + half], sem))
                out.append((ad_h.at[di], buf[1 + nh + half], sem))
                out.append((h_h.at[si], buf[1 + 2 * nh + half], sem))
            return out

        def fire(buf, g):
            sdidx = buf[0]
            pltpu.sync_copy(
                sd_h.at[pl.ds((wid * nbt + g) * 2 * nh, 2 * nh)], sdidx)
            for s, d, sem in _gathers(buf):
                pltpu.async_copy(s, d, sem)

        def drain(buf):
            for s, d, sem in _gathers(buf):
                pltpu.make_async_copy(s, d, sem).wait()

        def drain_scatter(buf):
            didx_s, ssem = buf[4 * nh + 1], buf[4 * nh + 3]
            for half in range(nh):
                pltpu.make_async_copy(buf[1 + 3 * nh + half],
                                      accum.at[didx_s.at[half]], ssem).wait()

        def compute_scatter(buf):
            sdidx = buf[0]
            didx_s, ssem = buf[4 * nh + 1], buf[4 * nh + 3]
            lane = lax.iota(jnp.int32, 16)
            reps = [jnp.where(lane < 8, 2 * j, 2 * j + 1) for j in range(nj)]

            for half in range(nh):
                as_v, ad_v = buf[1 + half], buf[1 + nh + half]
                h_v, msg_v = buf[1 + 2 * nh + half], buf[1 + 3 * nh + half]

                def do_edge(e):
                    ea = as_v[e] + ad_v[e]
                    ea = jnp.where(ea >= 0, ea, 0.2 * ea)
                    ex = jnp.exp(ea)
                    for j in range(nj):
                        hvec = h_v[e, pl.ds(16 * j, 16)]
                        rep = _dgather(ex, reps[j]) if nj > 1 else ex
                        msg_v[e, pl.ds(16 * j, 16)] = hvec * rep
                    msg_v[e, pl.ds(fw, 16)] = ex

                def edge_body(k, c2):
                    do_edge(2 * k)
                    do_edge(2 * k + 1)
                    return c2

                lax.fori_loop(0, BLK // 2, edge_body, 0)
                for t in range(BLK // 16):
                    didx_s[half, pl.ds(16 * t, 16)] = (
                        sdidx[2 * half + 1, pl.ds(16 * t, 16)])
                pltpu.async_copy(msg_v, accum.at[didx_s.at[half]],
                                 ssem, add=True)

        fire(bufs[0], 0)

        zmsg = bufs[0][1 + 3 * nh]
        zvec = jnp.zeros((16,), jnp.float32)

        def zrow(r, c2):
            for j in range(aw // 16):
                zmsg[r, pl.ds(16 * j, 16)] = zvec
            return c2

        lax.fori_loop(0, BLK, zrow, 0)
        for k in range(ROWS_PT // BLK):
            pltpu.sync_copy(zmsg, accum.at[pl.ds(lo + k * BLK, BLK)])
        plsc.subcore_barrier()

        fire(bufs[1], 1)
        drain(bufs[0])
        compute_scatter(bufs[0])
        fire(bufs[0], 2)
        drain(bufs[1])
        compute_scatter(bufs[1])

        def blk_body(i, carry):
            fire(bufs[1], 2 * i + 1)
            drain(bufs[0])
            drain_scatter(bufs[0])
            compute_scatter(bufs[0])
            fire(bufs[0], 2 * i + 2)
            drain(bufs[1])
            drain_scatter(bufs[1])
            compute_scatter(bufs[1])
            return carry

        lax.fori_loop(1, nbt // 2, blk_body, 0)
        drain(bufs[0])
        drain_scatter(bufs[0])
        drain_scatter(bufs[1])

        plsc.subcore_barrier()
        pltpu.sync_copy(accum.at[pl.ds(lo, ROWS_PT)],
                        out_h.at[cid, pl.ds(lo, ROWS_PT)])

    return edge_kernel


_edge_pass_l1 = _make_edge_pass(F1, AW1, 10496, 1)
_edge_pass_l2 = _make_edge_pass(C2, AW2, EPT, 2)


def _mid_body(p_ref, r_ref, b1_ref, w2_ref, as2_ref, ad2_ref,
              h2_ref, asr_ref, adr_ref):
    acc = p_ref[0] + p_ref[1]
    num = acc[:, :F1]
    den = acc[:, F1:F1 + H1]
    invr = jnp.dot(1.0 / (den + 1e-16), r_ref[...],
                   preferred_element_type=jnp.float32)
    hl = num * invr + b1_ref[...]
    hf = jnp.where(hl > 0, hl, jnp.exp(hl) - 1.0)
    h2 = jnp.dot(hf, w2_ref[...], preferred_element_type=jnp.float32)
    h2_ref[...] = h2
    asr_ref[...] = jnp.dot(h2, as2_ref[...], preferred_element_type=jnp.float32)
    adr_ref[...] = jnp.dot(h2, ad2_ref[...], preferred_element_type=jnp.float32)


def _stage3(part1, R8, b1_2d, W2, AS2, AD2):
    blk = NP // 8
    return pl.pallas_call(
        _mid_body,
        grid=(8,),
        in_specs=[
            pl.BlockSpec((2, blk, AW1), lambda i: (0, i, 0)),
            pl.BlockSpec((H1, F1), lambda i: (0, 0)),
            pl.BlockSpec((1, F1), lambda i: (0, 0)),
            pl.BlockSpec((F1, C2), lambda i: (0, 0)),
            pl.BlockSpec((C2, 16), lambda i: (0, 0)),
            pl.BlockSpec((C2, 16), lambda i: (0, 0)),
        ],
        out_specs=[
            pl.BlockSpec((blk, C2), lambda i: (i, 0)),
            pl.BlockSpec((blk, 16), lambda i: (i, 0)),
            pl.BlockSpec((blk, 16), lambda i: (i, 0)),
        ],
        out_shape=[
            jax.ShapeDtypeStruct((NP, C2), jnp.float32),
            jax.ShapeDtypeStruct((NP, 16), jnp.float32),
            jax.ShapeDtypeStruct((NP, 16), jnp.float32),
        ],
    )(part1, R8, b1_2d, W2, AS2, AD2)


def _fin_body(p_ref, b2_ref, out_ref):
    acc = p_ref[0] + p_ref[1]
    o = acc[:, :C2] / (acc[:, C2:2 * C2] + 1e-16) + b2_ref[...]
    m = jnp.max(o, axis=1, keepdims=True)
    out_ref[...] = o - m - jnp.log(jnp.sum(jnp.exp(o - m), axis=1,
                                           keepdims=True))


def _stage5(part2, b2_2d):
    blk = NP // 8
    return pl.pallas_call(
        _fin_body,
        grid=(8,),
        in_specs=[
            pl.BlockSpec((2, blk, AW2), lambda i: (0, i, 0)),
            pl.BlockSpec((1, C2), lambda i: (0, 0)),
        ],
        out_specs=pl.BlockSpec((blk, C2), lambda i: (i, 0)),
        out_shape=jax.ShapeDtypeStruct((NP, C2), jnp.float32),
    )(part2, b2_2d)


def kernel(x, edge_index, W1, a_s1, a_d1, b1, W2, a_s2, a_d2, b2):
    f32 = jnp.float32
    x_pad = jnp.pad(x, ((0, NP - N), (0, 0)))

    loops = jnp.arange(N, dtype=jnp.int32)
    pad = jnp.full((EALLOC - E - N,), N, dtype=jnp.int32)
    src = jnp.concatenate([edge_index[0].astype(jnp.int32), loops, pad])
    dst = jnp.concatenate([edge_index[1].astype(jnp.int32), loops, pad])
    sd = jnp.stack([src.reshape(-1, BLK), dst.reshape(-1, BLK)],
                   axis=1).reshape(-1, BLK)

    rows = jnp.arange(F1)
    AS1 = jnp.zeros((F1, 16), f32).at[rows, rows // C1].set(a_s1.reshape(F1))
    AD1 = jnp.zeros((F1, 16), f32).at[rows, rows // C1].set(a_d1.reshape(F1))

    h1, asrc1, adst1 = _stage1(x_pad, W1, AS1, AD1)

    part1 = _edge_pass_l1(sd, asrc1, adst1, h1)

    R8 = jnp.zeros((H1, F1), f32).at[jnp.arange(F1) // C1, jnp.arange(F1)].set(1.0)
    AS2 = jnp.broadcast_to(a_s2.reshape(C2, 1), (C2, 16)).astype(f32)
    AD2 = jnp.broadcast_to(a_d2.reshape(C2, 1), (C2, 16)).astype(f32)
    h2, asrc2, adst2 = _stage3(part1, R8, b1.reshape(1, F1), W2, AS2, AD2)

    part2 = _edge_pass_l2(sd, asrc2, adst2, h2)

    out = _stage5(part2, b2.reshape(1, C2))
    return out[:N]


# speedup vs baseline: 76.7579x; 1.0089x over previous
import functools

import jax
import jax.numpy as jnp
from jax import lax
from jax.experimental import pallas as pl
from jax.experimental.pallas import tpu as pltpu
from jax.experimental.pallas import tpu_sc as plsc

N = 10000
NP = 10240
E = 320000
EPP = 344064
EPT = EPP // 32
BLK = 128
SB = 256
EALLOC = EPP + SB
F_IN = 128
H1 = 8
C1 = 8
F1 = H1 * C1
C2 = 16
AW1 = 80
AW2 = 32
ROWS_PT = NP // 16


def _mm1_body(x_ref, w_ref, as_ref, ad_ref, h_ref, asrc_ref, adst_ref):
    h = jnp.dot(x_ref[...], w_ref[...], preferred_element_type=jnp.float32)
    h_ref[...] = h
    asrc_ref[...] = jnp.dot(h, as_ref[...], preferred_element_type=jnp.float32)
    adst_ref[...] = jnp.dot(h, ad_ref[...], preferred_element_type=jnp.float32)


def _stage1(x_pad, W1, AS1, AD1):
    blk = NP // 8
    return pl.pallas_call(
        _mm1_body,
        grid=(8,),
        in_specs=[
            pl.BlockSpec((blk, F_IN), lambda i: (i, 0)),
            pl.BlockSpec((F_IN, F1), lambda i: (0, 0)),
            pl.BlockSpec((F1, 16), lambda i: (0, 0)),
            pl.BlockSpec((F1, 16), lambda i: (0, 0)),
        ],
        out_specs=[
            pl.BlockSpec((blk, F1), lambda i: (i, 0)),
            pl.BlockSpec((blk, 16), lambda i: (i, 0)),
            pl.BlockSpec((blk, 16), lambda i: (i, 0)),
        ],
        out_shape=[
            jax.ShapeDtypeStruct((NP, F1), jnp.float32),
            jax.ShapeDtypeStruct((NP, 16), jnp.float32),
            jax.ShapeDtypeStruct((NP, 16), jnp.float32),
        ],
    )(x_pad, W1, AS1, AD1)


def _dgather(v, idx):
    return lax.gather(
        v, idx[:, None],
        lax.GatherDimensionNumbers(
            offset_dims=(), collapsed_slice_dims=(0,), start_index_map=(0,)),
        slice_sizes=(1,),
        mode=lax.GatherScatterMode.PROMISE_IN_BOUNDS)


def _make_edge_pass(fw, aw, ept, nh):
    nj = fw // 16
    mesh = plsc.VectorSubcoreMesh(core_axis_name="c", subcore_axis_name="s")

    nset = 4 * nh + 4
    buf_types = []
    for _ in range(2):
        buf_types += [pltpu.VMEM((2 * nh, BLK), jnp.int32)]
        buf_types += [pltpu.VMEM((BLK, 16), jnp.float32) for _ in range(nh)]
        buf_types += [pltpu.VMEM((BLK, 16), jnp.float32) for _ in range(nh)]
        buf_types += [pltpu.VMEM((BLK, fw), jnp.float32) for _ in range(nh)]
        buf_types += [pltpu.VMEM((BLK, aw), jnp.float32) for _ in range(nh)]
        buf_types += [
            pltpu.VMEM((nh, BLK), jnp.int32),
            pltpu.SemaphoreType.DMA,
            pltpu.SemaphoreType.DMA,
        ]

    @functools.partial(
        pl.kernel,
        mesh=mesh,
        compiler_params=pltpu.CompilerParams(use_tc_tiling_on_sc=False),
        out_type=jax.ShapeDtypeStruct((2, NP, aw), jnp.float32),
        scratch_types=buf_types + [pltpu.VMEM_SHARED((NP, aw), jnp.float32)],
    )
    def edge_kernel(sd_h, as_h, ad_h, h_h, out_h, *scr):
        bufs = (scr[0:nset], scr[nset:2 * nset])
        accum = scr[2 * nset]
        cid = lax.axis_index("c")
        sid = lax.axis_index("s")
        wid = cid * 16 + sid
        lo = sid * ROWS_PT
        nbt = ept // (nh * BLK)

        def _gathers(buf):
            sdidx, sem = buf[0], buf[4 * nh + 2]
            out = []
            for half in range(nh):
                si, di = sdidx.at[2 * half], sdidx.at[2 * half + 1]
                out.append((as_h.at[si], buf[1 + half], sem))
                out.append((ad_h.at[di], buf[1 + nh + half], sem))
                out.append((h_h.at[si], buf[1 + 2 * nh + half], sem))
            return out

        def fire(buf, g):
            sdidx = buf[0]
            pltpu.sync_copy(
                sd_h.at[pl.ds((wid * nbt + g) * 2 * nh, 2 * nh)], sdidx)
            for s, d, sem in _gathers(buf):
                pltpu.async_copy(s, d, sem)

        def drain(buf):
            for s, d, sem in _gathers(buf):
                pltpu.make_async_copy(s, d, sem).wait()

        def drain_scatter(buf):
            didx_s, ssem = buf[4 * nh + 1], buf[4 * nh + 3]
            for half in range(nh):
                pltpu.make_async_copy(buf[1 + 3 * nh + half],
                                      accum.at[didx_s.at[half]], ssem).wait()

        def compute_scatter(buf):
            sdidx = buf[0]
            didx_s, ssem = buf[4 * nh + 1], buf[4 * nh + 3]
            lane = lax.iota(jnp.int32, 16)
            reps = [jnp.where(lane < 8, 2 * j, 2 * j + 1) for j in range(nj)]

            for half in range(nh):
                as_v, ad_v = buf[1 + half], buf[1 + nh + half]
                h_v, msg_v = buf[1 + 2 * nh + half], buf[1 + 3 * nh + half]

                def do_edge(e):
                    ea = as_v[e] + ad_v[e]
                    ea = jnp.where(ea >= 0, ea, 0.2 * ea)
                    ex = jnp.exp(ea)
                    for j in range(nj):
                        hvec = h_v[e, pl.ds(16 * j, 16)]
                        rep = _dgather(ex, reps[j]) if nj > 1 else ex
                        msg_v[e, pl.ds(16 * j, 16)] = hvec * rep
                    msg_v[e, pl.ds(fw, 16)] = ex

                def edge_body(k, c2):
                    for u in range(4):
                        do_edge(4 * k + u)
                    return c2

                lax.fori_loop(0, BLK // 4, edge_body, 0)
                for t in range(BLK // 16):
                    didx_s[half, pl.ds(16 * t, 16)] = (
                        sdidx[2 * half + 1, pl.ds(16 * t, 16)])
                pltpu.async_copy(msg_v, accum.at[didx_s.at[half]],
                                 ssem, add=True)

        fire(bufs[0], 0)

        zmsg = bufs[0][1 + 3 * nh]
        zvec = jnp.zeros((16,), jnp.float32)

        def zrow(r, c2):
            for j in range(aw // 16):
                zmsg[r, pl.ds(16 * j, 16)] = zvec
            return c2

        lax.fori_loop(0, BLK, zrow, 0)
        for k in range(ROWS_PT // BLK):
            pltpu.sync_copy(zmsg, accum.at[pl.ds(lo + k * BLK, BLK)])
        plsc.subcore_barrier()

        fire(bufs[1], 1)
        drain(bufs[0])
        compute_scatter(bufs[0])
        fire(bufs[0], 2)
        drain(bufs[1])
        compute_scatter(bufs[1])

        def blk_body(i, carry):
            fire(bufs[1], 2 * i + 1)
            drain(bufs[0])
            drain_scatter(bufs[0])
            compute_scatter(bufs[0])
            fire(bufs[0], 2 * i + 2)
            drain(bufs[1])
            drain_scatter(bufs[1])
            compute_scatter(bufs[1])
            return carry

        lax.fori_loop(1, nbt // 2, blk_body, 0)
        drain(bufs[0])
        drain_scatter(bufs[0])
        drain_scatter(bufs[1])

        plsc.subcore_barrier()
        pltpu.sync_copy(accum.at[pl.ds(lo, ROWS_PT)],
                        out_h.at[cid, pl.ds(lo, ROWS_PT)])

    return edge_kernel


_edge_pass_l1 = _make_edge_pass(F1, AW1, 10496, 1)
_edge_pass_l2 = _make_edge_pass(C2, AW2, EPT, 2)


def _mid_body(p_ref, r_ref, b1_ref, w2_ref, as2_ref, ad2_ref,
              h2_ref, asr_ref, adr_ref):
    acc = p_ref[0] + p_ref[1]
    num = acc[:, :F1]
    den = acc[:, F1:F1 + H1]
    invr = jnp.dot(1.0 / (den + 1e-16), r_ref[...],
                   preferred_element_type=jnp.float32)
    hl = num * invr + b1_ref[...]
    hf = jnp.where(hl > 0, hl, jnp.exp(hl) - 1.0)
    h2 = jnp.dot(hf, w2_ref[...], preferred_element_type=jnp.float32)
    h2_ref[...] = h2
    asr_ref[...] = jnp.dot(h2, as2_ref[...], preferred_element_type=jnp.float32)
    adr_ref[...] = jnp.dot(h2, ad2_ref[...], preferred_element_type=jnp.float32)


def _stage3(part1, R8, b1_2d, W2, AS2, AD2):
    blk = NP // 8
    return pl.pallas_call(
        _mid_body,
        grid=(8,),
        in_specs=[
            pl.BlockSpec((2, blk, AW1), lambda i: (0, i, 0)),
            pl.BlockSpec((H1, F1), lambda i: (0, 0)),
            pl.BlockSpec((1, F1), lambda i: (0, 0)),
            pl.BlockSpec((F1, C2), lambda i: (0, 0)),
            pl.BlockSpec((C2, 16), lambda i: (0, 0)),
            pl.BlockSpec((C2, 16), lambda i: (0, 0)),
        ],
        out_specs=[
            pl.BlockSpec((blk, C2), lambda i: (i, 0)),
            pl.BlockSpec((blk, 16), lambda i: (i, 0)),
            pl.BlockSpec((blk, 16), lambda i: (i, 0)),
        ],
        out_shape=[
            jax.ShapeDtypeStruct((NP, C2), jnp.float32),
            jax.ShapeDtypeStruct((NP, 16), jnp.float32),
            jax.ShapeDtypeStruct((NP, 16), jnp.float32),
        ],
    )(part1, R8, b1_2d, W2, AS2, AD2)


def _fin_body(p_ref, b2_ref, out_ref):
    acc = p_ref[0] + p_ref[1]
    o = acc[:, :C2] / (acc[:, C2:2 * C2] + 1e-16) + b2_ref[...]
    m = jnp.max(o, axis=1, keepdims=True)
    out_ref[...] = o - m - jnp.log(jnp.sum(jnp.exp(o - m), axis=1,
                                           keepdims=True))


def _stage5(part2, b2_2d):
    blk = NP // 8
    return pl.pallas_call(
        _fin_body,
        grid=(8,),
        in_specs=[
            pl.BlockSpec((2, blk, AW2), lambda i: (0, i, 0)),
            pl.BlockSpec((1, C2), lambda i: (0, 0)),
        ],
        out_specs=pl.BlockSpec((blk, C2), lambda i: (i, 0)),
        out_shape=jax.ShapeDtypeStruct((NP, C2), jnp.float32),
    )(part2, b2_2d)


def kernel(x, edge_index, W1, a_s1, a_d1, b1, W2, a_s2, a_d2, b2):
    f32 = jnp.float32
    x_pad = jnp.pad(x, ((0, NP - N), (0, 0)))

    loops = jnp.arange(N, dtype=jnp.int32)
    pad = jnp.full((EALLOC - E - N,), N, dtype=jnp.int32)
    src = jnp.concatenate([edge_index[0].astype(jnp.int32), loops, pad])
    dst = jnp.concatenate([edge_index[1].astype(jnp.int32), loops, pad])
    sd = jnp.stack([src.reshape(-1, BLK), dst.reshape(-1, BLK)],
                   axis=1).reshape(-1, BLK)

    rows = jnp.arange(F1)
    AS1 = jnp.zeros((F1, 16), f32).at[rows, rows // C1].set(a_s1.reshape(F1))
    AD1 = jnp.zeros((F1, 16), f32).at[rows, rows // C1].set(a_d1.reshape(F1))

    h1, asrc1, adst1 = _stage1(x_pad, W1, AS1, AD1)

    part1 = _edge_pass_l1(sd, asrc1, adst1, h1)

    R8 = jnp.zeros((H1, F1), f32).at[jnp.arange(F1) // C1, jnp.arange(F1)].set(1.0)
    AS2 = jnp.broadcast_to(a_s2.reshape(C2, 1), (C2, 16)).astype(f32)
    AD2 = jnp.broadcast_to(a_d2.reshape(C2, 1), (C2, 16)).astype(f32)
    h2, asrc2, adst2 = _stage3(part1, R8, b1.reshape(1, F1), W2, AS2, AD2)

    part2 = _edge_pass_l2(sd, asrc2, adst2, h2)

    out = _stage5(part2, b2.reshape(1, C2))
    return out[:N]
